# baseline (speedup 1.0000x reference)
import functools

import jax
import jax.numpy as jnp
import numpy as np
from jax import lax
from jax.experimental import pallas as pl
from jax.experimental.pallas import tpu as pltpu
from jax.experimental.pallas import tpu_sc as plsc

N_FIELDS = 26
FIELD_SIZE = 100000
N_NUM = 13
D = 32
BATCH = 16384

NC, NS, L = 2, 16, 16
NW = NC * NS
ROWS_W = BATCH // NW
RB = 64
NBLK = ROWS_W // RB
CAT_BLK = RB * N_FIELDS
NUM_BLK = RB * N_NUM
PAT = L * N_FIELDS // 2


def _body(xc_hbm, xn_hbm, tab_hbm, w_hbm, b_hbm, pat_hbm, out_hbm,
          raw_v, cat_v, num_v, xn_v, w_v, b_v, pat_v, gsem, osem):
    wid = lax.axis_index("s") * NC + lax.axis_index("c")
    pltpu.sync_copy(w_hbm, w_v)
    pltpu.sync_copy(b_hbm, b_v)
    pltpu.sync_copy(pat_hbm, pat_v)

    def block(b, carry):
        base_row = wid * ROWS_W + b * RB
        pltpu.sync_copy(xc_hbm.at[pl.ds(base_row * N_FIELDS, CAT_BLK)], raw_v)
        pltpu.sync_copy(xn_hbm.at[pl.ds(base_row * N_NUM, NUM_BLK)], xn_v)

        def gath(i, c):
            idx = (raw_v[pl.ds(i * L, L)]
                   + pat_v[pl.ds(lax.rem(i, N_NUM) * L, L)])
            pltpu.async_copy(tab_hbm.at[idx], cat_v.at[pl.ds(i * L, L)], gsem)
            return c

        lax.fori_loop(0, CAT_BLK // L, gath, 0)

        def numr(r, c):
            for n in range(N_NUM):
                p = r * N_NUM + n
                xvec = plsc.load_gather(
                    xn_v, [jnp.zeros((L,), jnp.int32) + p])
                for h in range(2):
                    seg = pl.ds(n * D + h * L, L)
                    num_v[p, pl.ds(h * L, L)] = xvec * w_v[seg] + b_v[seg]
            return c

        lax.fori_loop(0, RB, numr, 0)

        pltpu.make_async_copy(tab_hbm.at[pl.ds(0, CAT_BLK)], cat_v, gsem).wait()

        def wr(r, c):
            row = base_row + r
            pltpu.async_copy(cat_v.at[pl.ds(r * N_FIELDS, N_FIELDS)],
                             out_hbm.at[row, pl.ds(0, N_FIELDS)], osem)
            pltpu.async_copy(num_v.at[pl.ds(r * N_NUM, N_NUM)],
                             out_hbm.at[row, pl.ds(N_FIELDS, N_NUM)], osem)
            return c

        lax.fori_loop(0, RB, wr, 0)
        pltpu.make_async_copy(tab_hbm.at[pl.ds(0, CAT_BLK)], cat_v, osem).wait()
        pltpu.make_async_copy(tab_hbm.at[pl.ds(0, NUM_BLK)], num_v, osem).wait()
        return carry

    lax.fori_loop(0, NBLK, block, 0)


@functools.partial(jax.jit)
def kernel(x_categ, x_numer, table, num_weights, num_biases):
    xc = x_categ.astype(jnp.int32).reshape(-1)
    xn = x_numer.reshape(-1)
    wf = num_weights.reshape(-1)
    bf = num_biases.reshape(-1)
    pat = jnp.asarray(
        (np.arange(PAT) % N_FIELDS).astype(np.int32) * FIELD_SIZE)

    run = pl.kernel(
        _body,
        out_type=jax.ShapeDtypeStruct((BATCH, N_FIELDS + N_NUM, D),
                                      jnp.float32),
        mesh=plsc.VectorSubcoreMesh(core_axis_name="c", subcore_axis_name="s"),
        compiler_params=pltpu.CompilerParams(use_tc_tiling_on_sc=False,
                                            needs_layout_passes=False),
        scratch_types=[
            pltpu.VMEM((CAT_BLK,), jnp.int32),
            pltpu.VMEM((CAT_BLK, D), jnp.float32),
            pltpu.VMEM((NUM_BLK, D), jnp.float32),
            pltpu.VMEM((NUM_BLK,), jnp.float32),
            pltpu.VMEM((N_NUM * D,), jnp.float32),
            pltpu.VMEM((N_NUM * D,), jnp.float32),
            pltpu.VMEM((PAT,), jnp.int32),
            pltpu.SemaphoreType.DMA,
            pltpu.SemaphoreType.DMA,
        ],
    )
    return run(xc, xn, table, wf, bf, pat)
